# baseline (speedup 1.0000x reference)
import jax
import jax.numpy as jnp
from jax.experimental import pallas as pl
from jax.experimental.pallas import tpu as pltpu

_B = 4
_N = 16384
_D = 128
_C = 64


def _kmpp_body(init_ref, data_ref, dataT_ref, out_ref, mind2_ref, x2_ref, sem):
    for b in range(_B):
        i0 = init_ref[b]
        pltpu.make_async_copy(
            data_ref.at[b, pl.ds(i0, 1), :],
            out_ref.at[b, pl.ds(0, 1), :],
            sem.at[b],
        ).start()
    for b in range(_B):
        dt = dataT_ref[b]
        x2_ref[b : b + 1, :] = jnp.sum(dt * dt, axis=0, keepdims=True)
    mind2_ref[:, :] = jnp.full((_B, _N), jnp.inf, dtype=jnp.float32)

    iota = jax.lax.broadcasted_iota(jnp.int32, (1, _N), 1)

    def step(i, carry):
        for b in range(_B):
            pltpu.make_async_copy(
                data_ref.at[b, pl.ds(0, 1), :],
                out_ref.at[b, pl.ds(0, 1), :],
                sem.at[b],
            ).wait()
            cprev = out_ref[b, pl.ds(i - 1, 1), :]
            dot = jax.lax.dot_general(
                cprev,
                dataT_ref[b],
                (((1,), (0,)), ((), ())),
                preferred_element_type=jnp.float32,
            )
            c2 = jnp.sum(cprev * cprev)
            d2 = jnp.maximum(x2_ref[b : b + 1, :] + c2 - 2.0 * dot, 0.0)
            m = jnp.minimum(mind2_ref[b : b + 1, :], d2)
            mind2_ref[b : b + 1, :] = m
            mx = jnp.max(m)
            idx = jnp.min(jnp.where(m == mx, iota, _N))
            pltpu.make_async_copy(
                data_ref.at[b, pl.ds(idx, 1), :],
                out_ref.at[b, pl.ds(i, 1), :],
                sem.at[b],
            ).start()
        return carry

    jax.lax.fori_loop(1, _C, step, 0)

    for b in range(_B):
        pltpu.make_async_copy(
            data_ref.at[b, pl.ds(0, 1), :],
            out_ref.at[b, pl.ds(0, 1), :],
            sem.at[b],
        ).wait()


def kernel(data):
    b, n, d = data.shape
    init_key = jax.random.key(42)
    init_idx = jax.random.randint(init_key, (b,), 0, n).astype(jnp.int32)
    dataT = jnp.swapaxes(data, 1, 2)
    return pl.pallas_call(
        _kmpp_body,
        out_shape=jax.ShapeDtypeStruct((b, _C, d), jnp.float32),
        in_specs=[
            pl.BlockSpec(memory_space=pltpu.SMEM),
            pl.BlockSpec(memory_space=pltpu.MemorySpace.HBM),
            pl.BlockSpec(memory_space=pltpu.VMEM),
        ],
        out_specs=pl.BlockSpec(memory_space=pltpu.VMEM),
        scratch_shapes=[
            pltpu.VMEM((_B, _N), jnp.float32),
            pltpu.VMEM((_B, _N), jnp.float32),
            pltpu.SemaphoreType.DMA((_B,)),
        ],
        compiler_params=pltpu.CompilerParams(
            vmem_limit_bytes=100 * 1024 * 1024,
        ),
    )(init_idx, data, dataT)


# speedup vs baseline: 1.1382x; 1.1382x over previous
import jax
import jax.numpy as jnp
from jax.experimental import pallas as pl
from jax.experimental.pallas import tpu as pltpu

_B = 4
_N = 16384
_D = 128
_C = 64


def _kmpp_body(init_ref, data_ref, dataT_ref, out_ref, mind2_ref, x2_ref, sem):
    for b in range(_B):
        i0 = init_ref[b]
        pltpu.make_async_copy(
            data_ref.at[b, pl.ds(i0, 1), :],
            out_ref.at[b, pl.ds(0, 1), :],
            sem.at[b],
        ).start()
    for b in range(_B):
        dt = dataT_ref[b]
        x2_ref[b : b + 1, :] = jnp.sum(dt * dt, axis=0, keepdims=True)
    mind2_ref[:, :] = jnp.full((_B, _N), jnp.inf, dtype=jnp.float32)

    iota = jax.lax.broadcasted_iota(jnp.int32, (1, _N), 1)

    def step(i, carry):
        for b in range(_B):
            pltpu.make_async_copy(
                data_ref.at[b, pl.ds(0, 1), :],
                out_ref.at[b, pl.ds(0, 1), :],
                sem.at[b],
            ).wait()
        dots = []
        cprevs = []
        for b in range(_B):
            cprev = out_ref[b, pl.ds(i - 1, 1), :]
            cprevs.append(cprev)
            dots.append(
                jax.lax.dot_general(
                    cprev,
                    dataT_ref[b],
                    (((1,), (0,)), ((), ())),
                    preferred_element_type=jnp.float32,
                )
            )
        for b in range(_B):
            cprev = cprevs[b]
            c2 = jnp.sum(cprev * cprev)
            d2 = jnp.maximum(x2_ref[b : b + 1, :] + c2 - 2.0 * dots[b], 0.0)
            m = jnp.minimum(mind2_ref[b : b + 1, :], d2)
            mind2_ref[b : b + 1, :] = m
            mx = jnp.max(m)
            idx = jnp.min(jnp.where(m == mx, iota, _N))
            pltpu.make_async_copy(
                data_ref.at[b, pl.ds(idx, 1), :],
                out_ref.at[b, pl.ds(i, 1), :],
                sem.at[b],
            ).start()
        return carry

    jax.lax.fori_loop(1, _C, step, 0)

    for b in range(_B):
        pltpu.make_async_copy(
            data_ref.at[b, pl.ds(0, 1), :],
            out_ref.at[b, pl.ds(0, 1), :],
            sem.at[b],
        ).wait()


def kernel(data):
    b, n, d = data.shape
    init_key = jax.random.key(42)
    init_idx = jax.random.randint(init_key, (b,), 0, n).astype(jnp.int32)
    dataT = jnp.swapaxes(data, 1, 2)
    return pl.pallas_call(
        _kmpp_body,
        out_shape=jax.ShapeDtypeStruct((b, _C, d), jnp.float32),
        in_specs=[
            pl.BlockSpec(memory_space=pltpu.SMEM),
            pl.BlockSpec(memory_space=pltpu.MemorySpace.HBM),
            pl.BlockSpec(memory_space=pltpu.VMEM),
        ],
        out_specs=pl.BlockSpec(memory_space=pltpu.VMEM),
        scratch_shapes=[
            pltpu.VMEM((_B, _N), jnp.float32),
            pltpu.VMEM((_B, _N), jnp.float32),
            pltpu.SemaphoreType.DMA((_B,)),
        ],
        compiler_params=pltpu.CompilerParams(
            vmem_limit_bytes=100 * 1024 * 1024,
        ),
    )(init_idx, data, dataT)


# speedup vs baseline: 1.2949x; 1.1377x over previous
import jax
import jax.numpy as jnp
from jax.experimental import pallas as pl
from jax.experimental.pallas import tpu as pltpu

_B = 4
_N = 16384
_D = 128
_C = 64


def _gather_row(dataT_ref, panel_ref, b, idx):
    start = pl.multiple_of((idx // 128) * 128, 128)
    panel = dataT_ref[b, :, pl.ds(start, 128)]
    panel_ref[b] = jnp.swapaxes(panel, 0, 1)
    return panel_ref[b, pl.ds(idx % 128, 1), :]


def _kmpp_body(init_ref, dataT_ref, out_ref, mind2_ref, x2_ref, panel_ref):
    for b in range(_B):
        dt = dataT_ref[b]
        x2_ref[b : b + 1, :] = jnp.sum(dt * dt, axis=0, keepdims=True)
    mind2_ref[:, :] = jnp.full((_B, _N), jnp.inf, dtype=jnp.float32)

    iota = jax.lax.broadcasted_iota(jnp.int32, (1, _N), 1)

    rows = []
    for b in range(_B):
        row = _gather_row(dataT_ref, panel_ref, b, init_ref[b])
        out_ref[b, 0:1, :] = row
        rows.append(row)

    def step(i, carry):
        dots = []
        for b in range(_B):
            dots.append(
                jax.lax.dot_general(
                    carry[b],
                    dataT_ref[b],
                    (((1,), (0,)), ((), ())),
                    preferred_element_type=jnp.float32,
                )
            )
        newrows = []
        for b in range(_B):
            cprev = carry[b]
            c2 = jnp.sum(cprev * cprev)
            d2 = jnp.maximum(x2_ref[b : b + 1, :] + c2 - 2.0 * dots[b], 0.0)
            m = jnp.minimum(mind2_ref[b : b + 1, :], d2)
            mind2_ref[b : b + 1, :] = m
            mx = jnp.max(m)
            idx = jnp.min(jnp.where(m == mx, iota, _N))
            row = _gather_row(dataT_ref, panel_ref, b, idx)
            out_ref[b, pl.ds(i, 1), :] = row
            newrows.append(row)
        return tuple(newrows)

    jax.lax.fori_loop(1, _C, step, tuple(rows))


def kernel(data):
    b, n, d = data.shape
    init_key = jax.random.key(42)
    init_idx = jax.random.randint(init_key, (b,), 0, n).astype(jnp.int32)
    dataT = jnp.swapaxes(data, 1, 2)
    return pl.pallas_call(
        _kmpp_body,
        out_shape=jax.ShapeDtypeStruct((b, _C, d), jnp.float32),
        in_specs=[
            pl.BlockSpec(memory_space=pltpu.SMEM),
            pl.BlockSpec(memory_space=pltpu.VMEM),
        ],
        out_specs=pl.BlockSpec(memory_space=pltpu.VMEM),
        scratch_shapes=[
            pltpu.VMEM((_B, _N), jnp.float32),
            pltpu.VMEM((_B, _N), jnp.float32),
            pltpu.VMEM((_B, 128, _D), jnp.float32),
        ],
        compiler_params=pltpu.CompilerParams(
            vmem_limit_bytes=100 * 1024 * 1024,
        ),
    )(init_idx, dataT)


# speedup vs baseline: 1.4333x; 1.1069x over previous
import jax
import jax.numpy as jnp
from jax.experimental import pallas as pl
from jax.experimental.pallas import tpu as pltpu

_B = 4
_N = 16384
_D = 128
_C = 64


def _gather_row(dataT_ref, panel_ref, b, idx):
    start = pl.multiple_of((idx // 128) * 128, 128)
    panel = dataT_ref[b, :, pl.ds(start, 128)]
    panel_ref[b] = jnp.swapaxes(panel, 0, 1)
    return panel_ref[b, pl.ds(idx % 128, 1), :]


def _kmpp_body(init_ref, dataT_ref, out_ref, mind2_ref, x2_ref, panel_ref):
    for b in range(_B):
        dt = dataT_ref[b]
        x2_ref[b : b + 1, :] = jnp.sum(dt * dt, axis=0, keepdims=True)
    mind2_ref[:, :] = jnp.full((_B, _N), jnp.inf, dtype=jnp.float32)

    iota = jax.lax.broadcasted_iota(jnp.int32, (1, _N), 1)

    rows = []
    for b in range(_B):
        row = _gather_row(dataT_ref, panel_ref, b, init_ref[b])
        out_ref[b, 0:1, :] = row
        rows.append(row)

    def step(i, carry):
        dots = []
        for b in range(_B):
            dots.append(
                jax.lax.dot_general(
                    carry[b],
                    dataT_ref[b],
                    (((1,), (0,)), ((), ())),
                    preferred_element_type=jnp.float32,
                )
            )
        newrows = []
        for b in range(_B):
            cprev = carry[b]
            c2 = jnp.sum(cprev * cprev)
            d2 = jnp.maximum(x2_ref[b : b + 1, :] + c2 - 2.0 * dots[b], 0.0)
            m = jnp.minimum(mind2_ref[b : b + 1, :], d2)
            mind2_ref[b : b + 1, :] = m
            mx = jnp.max(m)
            idx = jnp.min(jnp.where(m == mx, iota, _N))
            row = _gather_row(dataT_ref, panel_ref, b, idx)
            out_ref[b, pl.ds(i, 1), :] = row
            newrows.append(row)
        return tuple(newrows)

    def trip(t, carry):
        i = 1 + t * 3
        carry = step(i, carry)
        carry = step(i + 1, carry)
        carry = step(i + 2, carry)
        return carry

    jax.lax.fori_loop(0, (_C - 1) // 3, trip, tuple(rows))


def kernel(data):
    b, n, d = data.shape
    init_key = jax.random.key(42)
    init_idx = jax.random.randint(init_key, (b,), 0, n).astype(jnp.int32)
    dataT = jnp.swapaxes(data, 1, 2)
    return pl.pallas_call(
        _kmpp_body,
        out_shape=jax.ShapeDtypeStruct((b, _C, d), jnp.float32),
        in_specs=[
            pl.BlockSpec(memory_space=pltpu.SMEM),
            pl.BlockSpec(memory_space=pltpu.VMEM),
        ],
        out_specs=pl.BlockSpec(memory_space=pltpu.VMEM),
        scratch_shapes=[
            pltpu.VMEM((_B, _N), jnp.float32),
            pltpu.VMEM((_B, _N), jnp.float32),
            pltpu.VMEM((_B, 128, _D), jnp.float32),
        ],
        compiler_params=pltpu.CompilerParams(
            vmem_limit_bytes=100 * 1024 * 1024,
        ),
    )(init_idx, dataT)
